# baseline (speedup 1.0000x reference)
import functools

import jax
import jax.numpy as jnp
from jax import lax
from jax.experimental import pallas as pl
from jax.experimental.pallas import tpu as pltpu
from jax.experimental.pallas import tpu_sc as plsc

BATCH = 4096
SEQ_LEN = 200
EMBED_DIM = 64
PADDED_DIM = 128

TOT = BATCH * SEQ_LEN
IDX_MINOR = 128
IDX_ROWS = TOT // IDX_MINOR

NUM_WORKERS = 32
ROWS_PER_W = IDX_ROWS // NUM_WORKERS
NB = 8
SUB = 4
G = ROWS_PER_W // NB

_mesh = plsc.VectorSubcoreMesh(core_axis_name="c", subcore_axis_name="s")


BPW = BATCH // NUM_WORKERS


@functools.partial(
    pl.kernel,
    mesh=_mesh,
    out_type=jax.ShapeDtypeStruct((SEQ_LEN, EMBED_DIM, BATCH), jnp.float32),
    scratch_types=[
        pltpu.VMEM((2, 8, BPW), jnp.int32),
        pltpu.VMEM((2, BPW, PADDED_DIM), jnp.float32),
        pltpu.VMEM((EMBED_DIM, BPW), jnp.float32),
        pltpu.SemaphoreType.DMA,
        pltpu.SemaphoreType.DMA,
    ],
    compiler_params=pltpu.CompilerParams(needs_layout_passes=False),
)
def _sc_gather(idx_hbm, table_hbm, out_hbm, idx_v, rows_v, tq, sg0, sg1):
    wid = lax.axis_index("s") * 2 + lax.axis_index("c")
    b0 = wid * BPW
    sem_g = (sg0, sg1)

    def load_idx_block(blk):
        pltpu.sync_copy(
            idx_hbm.at[pl.ds(blk * 8, 8), pl.ds(b0, BPW)], idx_v.at[blk & 1]
        )

    def fire_gather(l, p):
        pltpu.async_copy(
            table_hbm.at[idx_v.at[(l // 8) & 1, l % 8]], rows_v.at[p], sem_g[p]
        )

    def drain_gather(p):
        pltpu.make_async_copy(
            table_hbm.at[pl.ds(0, BPW)], rows_v.at[p], sem_g[p]
        ).wait()

    def xpose_store(l, p):
        rv = rows_v.at[p]

        iotas = [lax.iota(jnp.int32, 16) + j0 * 16 for j0 in range(BPW // 16)]
        for d in range(EMBED_DIM):
            dvec = jnp.full((16,), d, jnp.int32)
            for j0 in range(BPW // 16):
                vals = plsc.load_gather(rv, [iotas[j0], dvec])
                tq[d, pl.ds(j0 * 16, 16)] = vals
        pltpu.sync_copy(tq, out_hbm.at[l, :, pl.ds(b0, BPW)])

    load_idx_block(0)
    fire_gather(0, 0)

    def body(i, _):
        l0 = 2 * i
        fire_gather(l0 + 1, 1)
        drain_gather(0)
        xpose_store(l0, 0)

        @pl.when((i % 4 == 3) & (i < SEQ_LEN // 2 - 1))
        def _():
            load_idx_block((i + 1) // 4)

        @pl.when(i < SEQ_LEN // 2 - 1)
        def _():
            fire_gather(l0 + 2, 0)

        drain_gather(1)
        xpose_store(l0 + 1, 1)
        return 0

    lax.fori_loop(0, SEQ_LEN // 2, body, 0)


def _mask_body(idx_ref, mask_ref):
    mask_ref[...] = (idx_ref[...] != 0).astype(jnp.int32)


_mask_call = pl.pallas_call(
    _mask_body,
    out_shape=jax.ShapeDtypeStruct((BATCH, SEQ_LEN), jnp.int32),
)

VOCAB = 1000000
VOCAB_PAD = 1000064
TBLK = 4096


def _xpose_body(src_ref, dst_ref):
    eye = jnp.eye(EMBED_DIM, dtype=jnp.float32)
    xt = jax.lax.dot_general(
        src_ref[...], eye, (((0,), (0,)), ((), ())),
        preferred_element_type=jnp.float32,
        precision=jax.lax.Precision.HIGHEST,
    )
    dst_ref[...] = jnp.concatenate(
        [xt, jnp.zeros((TBLK, PADDED_DIM - EMBED_DIM), jnp.float32)], axis=1
    )


_xpose_call = pl.pallas_call(
    _xpose_body,
    grid=((VOCAB + TBLK - 1) // TBLK,),
    in_specs=[pl.BlockSpec((EMBED_DIM, TBLK), lambda i: (0, i))],
    out_specs=pl.BlockSpec((TBLK, PADDED_DIM), lambda i: (i, 0)),
    out_shape=jax.ShapeDtypeStruct((VOCAB_PAD, PADDED_DIM), jnp.float32),
)


def kernel(indices, table):
    table_p = _xpose_call(table.T)
    rows_t = _sc_gather(indices.T, table_p)
    outputs = rows_t.transpose(2, 0, 1)
    mask = _mask_call(indices)
    return outputs, mask


# speedup vs baseline: 2.2659x; 2.2659x over previous
import functools

import jax
import jax.numpy as jnp
from jax import lax
from jax.experimental import pallas as pl
from jax.experimental.pallas import tpu as pltpu
from jax.experimental.pallas import tpu_sc as plsc

BATCH = 4096
SEQ_LEN = 200
EMBED_DIM = 64
PADDED_DIM = 128

TOT = BATCH * SEQ_LEN
IDX_MINOR = 128
IDX_ROWS = TOT // IDX_MINOR

NUM_WORKERS = 32
ROWS_PER_W = IDX_ROWS // NUM_WORKERS
NB = 8
SUB = 4
G = ROWS_PER_W // NB

_mesh = plsc.VectorSubcoreMesh(core_axis_name="c", subcore_axis_name="s")


@functools.partial(
    pl.kernel,
    mesh=_mesh,
    out_type=jax.ShapeDtypeStruct((IDX_ROWS, IDX_MINOR, PADDED_DIM), jnp.float32),
    scratch_types=[
        pltpu.VMEM((NB, IDX_MINOR), jnp.int32),
        pltpu.VMEM((SUB, IDX_MINOR, PADDED_DIM), jnp.float32),
        pltpu.SemaphoreType.DMA,
    ],
)
def _sc_gather(idx_hbm, table_hbm, out_hbm, idx_v, rows_v, sem):
    wid = lax.axis_index("s") * 2 + lax.axis_index("c")
    row0 = wid * ROWS_PER_W

    def body(g, _):
        r = row0 + g * NB
        pltpu.sync_copy(idx_hbm.at[pl.ds(r, NB), :], idx_v)
        for h in range(NB // SUB):
            handles = [
                pltpu.async_copy(
                    table_hbm.at[idx_v.at[h * SUB + j]], rows_v.at[j], sem
                )
                for j in range(SUB)
            ]
            for hd in handles:
                hd.wait()
            pltpu.sync_copy(rows_v, out_hbm.at[pl.ds(r + h * SUB, SUB)])
        return 0

    lax.fori_loop(0, G, body, 0)


def _mask_body(idx_ref, mask_ref):
    mask_ref[...] = (idx_ref[...] != 0).astype(jnp.int32)


_mask_call = pl.pallas_call(
    _mask_body,
    out_shape=jax.ShapeDtypeStruct((BATCH, SEQ_LEN), jnp.int32),
)

VOCAB = 1000000
VOCAB_PAD = 1000064
TBLK = 4096


def _xpose_body(src_ref, dst_ref):
    eye = jnp.eye(EMBED_DIM, dtype=jnp.float32)
    xt = jax.lax.dot_general(
        src_ref[...], eye, (((0,), (0,)), ((), ())),
        preferred_element_type=jnp.float32,
    )
    dst_ref[...] = jnp.concatenate(
        [xt, jnp.zeros((TBLK, PADDED_DIM - EMBED_DIM), jnp.float32)], axis=1
    )


_xpose_call = pl.pallas_call(
    _xpose_body,
    grid=((VOCAB + TBLK - 1) // TBLK,),
    in_specs=[pl.BlockSpec((EMBED_DIM, TBLK), lambda i: (0, i))],
    out_specs=pl.BlockSpec((TBLK, PADDED_DIM), lambda i: (i, 0)),
    out_shape=jax.ShapeDtypeStruct((VOCAB_PAD, PADDED_DIM), jnp.float32),
)


def kernel(indices, table):
    table_p = _xpose_call(table.T)
    idx2d = indices.reshape(IDX_ROWS, IDX_MINOR)
    rows = _sc_gather(idx2d, table_p)
    outputs = rows[:, :, :EMBED_DIM].reshape(BATCH, SEQ_LEN, EMBED_DIM)
    mask = _mask_call(indices)
    return outputs, mask


# speedup vs baseline: 2.2830x; 1.0076x over previous
import functools

import jax
import jax.numpy as jnp
from jax import lax
from jax.experimental import pallas as pl
from jax.experimental.pallas import tpu as pltpu
from jax.experimental.pallas import tpu_sc as plsc

BATCH = 4096
SEQ_LEN = 200
EMBED_DIM = 64
PADDED_DIM = 128

TOT = BATCH * SEQ_LEN
IDX_MINOR = 128
IDX_ROWS = TOT // IDX_MINOR

NUM_WORKERS = 32
ROWS_PER_W = IDX_ROWS // NUM_WORKERS
NB = 8
SUB = 4
G = ROWS_PER_W // NB

_mesh = plsc.VectorSubcoreMesh(core_axis_name="c", subcore_axis_name="s")


@functools.partial(
    pl.kernel,
    mesh=_mesh,
    out_type=jax.ShapeDtypeStruct((IDX_ROWS, IDX_MINOR, PADDED_DIM), jnp.float32),
    scratch_types=[
        pltpu.VMEM((NB, IDX_MINOR), jnp.int32),
        pltpu.VMEM((SUB, IDX_MINOR, PADDED_DIM), jnp.float32),
        pltpu.SemaphoreType.DMA,
    ],
)
def _sc_gather(idx_hbm, table_hbm, out_hbm, idx_v, rows_v, sem):
    wid = lax.axis_index("s") * 2 + lax.axis_index("c")
    row0 = wid * ROWS_PER_W

    def body(g, _):
        r = row0 + g * NB
        pltpu.sync_copy(idx_hbm.at[pl.ds(r, NB), :], idx_v)
        for h in range(NB // SUB):
            handles = [
                pltpu.async_copy(
                    table_hbm.at[idx_v.at[h * SUB + j]], rows_v.at[j], sem
                )
                for j in range(SUB)
            ]
            for hd in handles:
                hd.wait()
            pltpu.sync_copy(rows_v, out_hbm.at[pl.ds(r + h * SUB, SUB)])
        return 0

    lax.fori_loop(0, G, body, 0)


def _mask_body(idx_ref, mask_ref):
    mask_ref[...] = (idx_ref[...] != 0).astype(jnp.int32)


_mask_call = pl.pallas_call(
    _mask_body,
    out_shape=jax.ShapeDtypeStruct((BATCH, SEQ_LEN), jnp.int32),
)

VOCAB = 1000000
VOCAB_PAD = 1000064
TBLK = 4096


def _xpose_body(src_ref, dst_ref):
    xt = src_ref[...].T
    dst_ref[...] = jnp.concatenate(
        [xt, jnp.zeros((TBLK, PADDED_DIM - EMBED_DIM), jnp.float32)], axis=1
    )


_xpose_call = pl.pallas_call(
    _xpose_body,
    grid=((VOCAB + TBLK - 1) // TBLK,),
    in_specs=[pl.BlockSpec((EMBED_DIM, TBLK), lambda i: (0, i))],
    out_specs=pl.BlockSpec((TBLK, PADDED_DIM), lambda i: (i, 0)),
    out_shape=jax.ShapeDtypeStruct((VOCAB_PAD, PADDED_DIM), jnp.float32),
)


def kernel(indices, table):
    table_p = _xpose_call(table.T)
    idx2d = indices.reshape(IDX_ROWS, IDX_MINOR)
    rows = _sc_gather(idx2d, table_p)
    outputs = rows[:, :, :EMBED_DIM].reshape(BATCH, SEQ_LEN, EMBED_DIM)
    mask = _mask_call(indices)
    return outputs, mask


# speedup vs baseline: 2.3919x; 1.0477x over previous
import functools

import jax
import jax.numpy as jnp
from jax import lax
from jax.experimental import pallas as pl
from jax.experimental.pallas import tpu as pltpu
from jax.experimental.pallas import tpu_sc as plsc

BATCH = 4096
SEQ_LEN = 200
EMBED_DIM = 64
PADDED_DIM = 128

TOT = BATCH * SEQ_LEN
IDX_MINOR = 128
IDX_ROWS = TOT // IDX_MINOR

NUM_WORKERS = 32
ROWS_PER_W = IDX_ROWS // NUM_WORKERS
NB = 8
SUB = 4
G = ROWS_PER_W // NB

_mesh = plsc.VectorSubcoreMesh(core_axis_name="c", subcore_axis_name="s")


ST = 2
NSTAGE = ROWS_PER_W // ST
NITER = NSTAGE // 2


@functools.partial(
    pl.kernel,
    mesh=_mesh,
    out_type=jax.ShapeDtypeStruct((IDX_ROWS, IDX_MINOR, PADDED_DIM), jnp.float32),
    scratch_types=[
        pltpu.VMEM((2, NB, IDX_MINOR), jnp.int32),
        pltpu.VMEM((2, ST, IDX_MINOR, PADDED_DIM), jnp.float32),
        pltpu.SemaphoreType.DMA,
        pltpu.SemaphoreType.DMA,
        pltpu.SemaphoreType.DMA,
        pltpu.SemaphoreType.DMA,
    ],
)
def _sc_gather(idx_hbm, table_hbm, out_hbm, idx_v, rows_v, sg0, sg1, ss0, ss1):
    wid = lax.axis_index("s") * 2 + lax.axis_index("c")
    row0 = wid * ROWS_PER_W
    sem_g = (sg0, sg1)
    sem_st = (ss0, ss1)

    def load_idx_chunk(c):
        pltpu.sync_copy(idx_hbm.at[pl.ds(row0 + c * NB, NB), :], idx_v.at[c & 1])

    def fire_stage(s, b):
        c = s // (NB // ST)
        for j in range(ST):
            pltpu.async_copy(
                table_hbm.at[idx_v.at[c & 1, (s * ST) % NB + j]],
                rows_v.at[b].at[j],
                sem_g[b],
            )

    def drain_gathers(b):
        pltpu.make_async_copy(
            out_hbm.at[pl.ds(0, ST)], rows_v.at[b], sem_g[b]
        ).wait()

    def store_stage(s, b):
        pltpu.async_copy(
            rows_v.at[b], out_hbm.at[pl.ds(row0 + s * ST, ST)], sem_st[b]
        )

    def drain_store(b):
        pltpu.make_async_copy(
            out_hbm.at[pl.ds(0, ST)], rows_v.at[b], sem_st[b]
        ).wait()

    load_idx_chunk(0)
    fire_stage(0, 0)

    def body(i, _):
        s0 = 2 * i

        @pl.when(i > 0)
        def _():
            drain_store(1)

        fire_stage(s0 + 1, 1)
        drain_gathers(0)
        store_stage(s0, 0)

        @pl.when(((i % 2) == 1) & (i < NITER - 1))
        def _():
            load_idx_chunk((i + 1) // 2)

        @pl.when(i < NITER - 1)
        def _():
            drain_store(0)
            fire_stage(s0 + 2, 0)

        drain_gathers(1)
        store_stage(s0 + 1, 1)
        return 0

    lax.fori_loop(0, NITER, body, 0)
    drain_store(0)
    drain_store(1)


def _mask_body(idx_ref, mask_ref):
    mask_ref[...] = (idx_ref[...] != 0).astype(jnp.int32)


_mask_call = pl.pallas_call(
    _mask_body,
    out_shape=jax.ShapeDtypeStruct((BATCH, SEQ_LEN), jnp.int32),
)

VOCAB = 1000000
VOCAB_PAD = 1000064
TBLK = 4096


def _xpose_body(src_ref, dst_ref):
    xt = src_ref[...].T
    dst_ref[...] = jnp.concatenate(
        [xt, jnp.zeros((TBLK, PADDED_DIM - EMBED_DIM), jnp.float32)], axis=1
    )


_xpose_call = pl.pallas_call(
    _xpose_body,
    grid=((VOCAB + TBLK - 1) // TBLK,),
    in_specs=[pl.BlockSpec((EMBED_DIM, TBLK), lambda i: (0, i))],
    out_specs=pl.BlockSpec((TBLK, PADDED_DIM), lambda i: (i, 0)),
    out_shape=jax.ShapeDtypeStruct((VOCAB_PAD, PADDED_DIM), jnp.float32),
)


def kernel(indices, table):
    table_p = _xpose_call(table.T)
    idx2d = indices.reshape(IDX_ROWS, IDX_MINOR)
    rows = _sc_gather(idx2d, table_p)
    outputs = rows[:, :, :EMBED_DIM].reshape(BATCH, SEQ_LEN, EMBED_DIM)
    mask = _mask_call(indices)
    return outputs, mask


# speedup vs baseline: 2.6246x; 1.0973x over previous
import functools

import jax
import jax.numpy as jnp
from jax import lax
from jax.experimental import pallas as pl
from jax.experimental.pallas import tpu as pltpu
from jax.experimental.pallas import tpu_sc as plsc

BATCH = 4096
SEQ_LEN = 200
EMBED_DIM = 64
PADDED_DIM = 128

TOT = BATCH * SEQ_LEN
IDX_MINOR = 128
IDX_ROWS = TOT // IDX_MINOR

NUM_WORKERS = 32
ROWS_PER_W = IDX_ROWS // NUM_WORKERS
NB = 8
SUB = 4
G = ROWS_PER_W // NB

_mesh = plsc.VectorSubcoreMesh(core_axis_name="c", subcore_axis_name="s")


ST = 2
NSTAGE = ROWS_PER_W // ST
NITER = NSTAGE // 2


@functools.partial(
    pl.kernel,
    mesh=_mesh,
    out_type=jax.ShapeDtypeStruct((IDX_ROWS, IDX_MINOR, PADDED_DIM), jnp.float32),
    scratch_types=[
        pltpu.VMEM((2, NB, IDX_MINOR), jnp.int32),
        pltpu.VMEM((2, ST, IDX_MINOR, PADDED_DIM), jnp.float32),
        pltpu.SemaphoreType.DMA,
        pltpu.SemaphoreType.DMA,
        pltpu.SemaphoreType.DMA,
        pltpu.SemaphoreType.DMA,
    ],
)
def _sc_gather(idx_hbm, table_hbm, out_hbm, idx_v, rows_v, sg0, sg1, ss0, ss1):
    wid = lax.axis_index("s") * 2 + lax.axis_index("c")
    row0 = wid * ROWS_PER_W
    sem_g = (sg0, sg1)
    sem_st = (ss0, ss1)

    def load_idx_chunk(c):
        pltpu.sync_copy(idx_hbm.at[pl.ds(row0 + c * NB, NB), :], idx_v.at[c & 1])

    def fire_stage(s, b):
        c = s // (NB // ST)
        for j in range(ST):
            pltpu.async_copy(
                table_hbm.at[idx_v.at[c & 1, (s * ST) % NB + j]],
                rows_v.at[b].at[j],
                sem_g[b],
            )

    def drain_gathers(b):
        pltpu.make_async_copy(
            out_hbm.at[pl.ds(0, ST)], rows_v.at[b], sem_g[b]
        ).wait()

    def store_stage(s, b):
        pltpu.async_copy(
            rows_v.at[b], out_hbm.at[pl.ds(row0 + s * ST, ST)], sem_st[b]
        )

    def drain_store(b):
        pltpu.make_async_copy(
            out_hbm.at[pl.ds(0, ST)], rows_v.at[b], sem_st[b]
        ).wait()

    load_idx_chunk(0)
    fire_stage(0, 0)

    def body(i, _):
        s0 = 2 * i

        @pl.when(i > 0)
        def _():
            drain_store(1)

        fire_stage(s0 + 1, 1)
        drain_gathers(0)
        store_stage(s0, 0)

        @pl.when(((i % 2) == 1) & (i < NITER - 1))
        def _():
            load_idx_chunk((i + 1) // 2)

        @pl.when(i < NITER - 1)
        def _():
            drain_store(0)
            fire_stage(s0 + 2, 0)

        drain_gathers(1)
        store_stage(s0 + 1, 1)
        return 0

    lax.fori_loop(0, NITER, body, 0)
    drain_store(0)
    drain_store(1)


def _mask_body(idx_ref, mask_ref):
    mask_ref[...] = (idx_ref[...] != 0).astype(jnp.int32)


_mask_call = pl.pallas_call(
    _mask_body,
    out_shape=jax.ShapeDtypeStruct((BATCH, SEQ_LEN), jnp.int32),
)

VOCAB = 1000000
VOCAB_PAD = 1000064
TBLK = 8192


def _xpose_body(src_ref, dst_ref):
    xt = src_ref[...].T
    dst_ref[...] = jnp.concatenate(
        [xt, jnp.zeros((TBLK, PADDED_DIM - EMBED_DIM), jnp.float32)], axis=1
    )


_xpose_call = pl.pallas_call(
    _xpose_body,
    grid=((VOCAB + TBLK - 1) // TBLK,),
    in_specs=[pl.BlockSpec((EMBED_DIM, TBLK), lambda i: (0, i))],
    out_specs=pl.BlockSpec((TBLK, PADDED_DIM), lambda i: (i, 0)),
    out_shape=jax.ShapeDtypeStruct((VOCAB_PAD, PADDED_DIM), jnp.float32),
)


def kernel(indices, table):
    table_p = _xpose_call(table.T)
    idx2d = indices.reshape(IDX_ROWS, IDX_MINOR)
    rows = _sc_gather(idx2d, table_p)
    outputs = rows[:, :, :EMBED_DIM].reshape(BATCH, SEQ_LEN, EMBED_DIM)
    mask = _mask_call(indices)
    return outputs, mask


# speedup vs baseline: 2.6875x; 1.0239x over previous
import functools

import jax
import jax.numpy as jnp
from jax import lax
from jax.experimental import pallas as pl
from jax.experimental.pallas import tpu as pltpu
from jax.experimental.pallas import tpu_sc as plsc

BATCH = 4096
SEQ_LEN = 200
EMBED_DIM = 64
PADDED_DIM = 128

TOT = BATCH * SEQ_LEN
IDX_MINOR = 128
IDX_ROWS = TOT // IDX_MINOR

NUM_WORKERS = 32
ROWS_PER_W = IDX_ROWS // NUM_WORKERS
NB = 8
SUB = 4
G = ROWS_PER_W // NB

_mesh = plsc.VectorSubcoreMesh(core_axis_name="c", subcore_axis_name="s")


ST = 2
NSTAGE = ROWS_PER_W // ST
NITER = NSTAGE // 2


@functools.partial(
    pl.kernel,
    mesh=_mesh,
    out_type=jax.ShapeDtypeStruct((IDX_ROWS, IDX_MINOR, PADDED_DIM), jnp.float32),
    scratch_types=[
        pltpu.VMEM((2, NB, IDX_MINOR), jnp.int32),
        pltpu.VMEM((2, ST, IDX_MINOR, PADDED_DIM), jnp.float32),
        pltpu.SemaphoreType.DMA,
        pltpu.SemaphoreType.DMA,
        pltpu.SemaphoreType.DMA,
        pltpu.SemaphoreType.DMA,
    ],
)
def _sc_gather(idx_hbm, table_hbm, out_hbm, idx_v, rows_v, sg0, sg1, ss0, ss1):
    wid = lax.axis_index("s") * 2 + lax.axis_index("c")
    row0 = wid * ROWS_PER_W
    sem_g = (sg0, sg1)
    sem_st = (ss0, ss1)

    def load_idx_chunk(c):
        pltpu.sync_copy(idx_hbm.at[pl.ds(row0 + c * NB, NB), :], idx_v.at[c & 1])

    def fire_stage(s, b):
        c = s // (NB // ST)
        for j in range(ST):
            pltpu.async_copy(
                table_hbm.at[idx_v.at[c & 1, (s * ST) % NB + j]],
                rows_v.at[b].at[j],
                sem_g[b],
            )

    def drain_gathers(b):
        pltpu.make_async_copy(
            out_hbm.at[pl.ds(0, ST)], rows_v.at[b], sem_g[b]
        ).wait()

    def store_stage(s, b):
        pltpu.async_copy(
            rows_v.at[b], out_hbm.at[pl.ds(row0 + s * ST, ST)], sem_st[b]
        )

    def drain_store(b):
        pltpu.make_async_copy(
            out_hbm.at[pl.ds(0, ST)], rows_v.at[b], sem_st[b]
        ).wait()

    load_idx_chunk(0)
    fire_stage(0, 0)

    def body(i, _):
        s0 = 2 * i

        @pl.when(i > 0)
        def _():
            drain_store(1)

        fire_stage(s0 + 1, 1)
        drain_gathers(0)
        store_stage(s0, 0)

        @pl.when(((i % 2) == 1) & (i < NITER - 1))
        def _():
            load_idx_chunk((i + 1) // 2)

        @pl.when(i < NITER - 1)
        def _():
            drain_store(0)
            fire_stage(s0 + 2, 0)

        drain_gathers(1)
        store_stage(s0 + 1, 1)
        return 0

    lax.fori_loop(0, NITER, body, 0)
    drain_store(0)
    drain_store(1)


def _mask_body(idx_ref, mask_ref):
    mask_ref[...] = (idx_ref[...] != 0).astype(jnp.int32)


_mask_call = pl.pallas_call(
    _mask_body,
    out_shape=jax.ShapeDtypeStruct((BATCH, SEQ_LEN), jnp.int32),
)

VOCAB = 1000000
VOCAB_PAD = 1000064
TBLK = 16384


def _xpose_body(src_ref, dst_ref):
    xt = src_ref[...].T
    dst_ref[...] = jnp.concatenate(
        [xt, jnp.zeros((TBLK, PADDED_DIM - EMBED_DIM), jnp.float32)], axis=1
    )


_xpose_call = pl.pallas_call(
    _xpose_body,
    grid=((VOCAB + TBLK - 1) // TBLK,),
    in_specs=[pl.BlockSpec((EMBED_DIM, TBLK), lambda i: (0, i))],
    out_specs=pl.BlockSpec((TBLK, PADDED_DIM), lambda i: (i, 0)),
    out_shape=jax.ShapeDtypeStruct((VOCAB_PAD, PADDED_DIM), jnp.float32),
)


def kernel(indices, table):
    table_p = _xpose_call(table.T)
    idx2d = indices.reshape(IDX_ROWS, IDX_MINOR)
    rows = _sc_gather(idx2d, table_p)
    outputs = rows[:, :, :EMBED_DIM].reshape(BATCH, SEQ_LEN, EMBED_DIM)
    mask = _mask_call(indices)
    return outputs, mask
